# baseline (speedup 1.0000x reference)
import functools

import jax
import jax.numpy as jnp
from jax import lax
from jax.experimental import pallas as pl
from jax.experimental.pallas import tpu as pltpu
from jax.experimental.pallas import tpu_sc as plsc

B = 16384
L = 50
D = 64
LANES = 16

_info = plsc.get_sparse_core_info()
NC = _info.num_cores
NS = _info.num_subcores
NW = NC * NS
ROWS_PER_W = B // NW
C = 16
NCHUNK = ROWS_PER_W // C


MPAD = 64


def _body(x_hbm, mask_hbm, table_hbm, out_hbm,
          idx_v0, idx_v1, mask_v0, mask_v1, rows_v0, rows_v1, out_v0, out_v1,
          gsem0, gsem1, osem0, osem1):
    wid = lax.axis_index("s") * NC + lax.axis_index("c")
    row0 = wid * ROWS_PER_W
    idx_v = (idx_v0, idx_v1)
    mask_v = (mask_v0, mask_v1)
    rows_v = (rows_v0, rows_v1)
    out_v = (out_v0, out_v1)
    gsem = (gsem0, gsem1)
    osem = (osem0, osem1)

    def fetch(ch, s):
        base = row0 + ch * C
        off = pl.multiple_of(base * L, 8)
        pltpu.sync_copy(x_hbm.at[pl.ds(off, C * L)], idx_v[s])
        pltpu.sync_copy(mask_hbm.at[pl.ds(base, C), :], mask_v[s])
        for k in range(C * L // LANES):
            v = idx_v[s][pl.ds(k * LANES, LANES)]
            o = v & (TW - 1)
            idx_v[s][pl.ds(k * LANES, LANES)] = (
                (v - o) + ((o & (TW // 4 - 1)) << 2) + (o >> QSH)
            )
        pltpu.async_copy(table_hbm.at[idx_v[s]], rows_v[s], gsem[s])

    def compute(ch, s):
        base = row0 + ch * C
        pltpu.make_async_copy(table_hbm.at[idx_v[s]], rows_v[s], gsem[s]).wait()

        hi_mask = jnp.full((LANES,), -65536, jnp.int32)

        def row(b, inner):
            mv = [mask_v[s][b, pl.ds(k * LANES, LANES)] for k in range(MPAD // LANES)]
            acc2 = [
                [jnp.zeros((LANES,), jnp.float32) for _ in range(D // LANES)]
                for _ in range(2)
            ]
            cnt2 = [jnp.float32(0.0), jnp.float32(0.0)]
            for l in range(L):
                m = mv[l // LANES][l % LANES]
                p = l & 1
                cnt2[p] = cnt2[p] + m
                for h in range(PW // LANES):
                    w = rows_v[s][b * L + l, pl.ds(h * LANES, LANES)]
                    lo = lax.bitcast_convert_type(w << 16, jnp.float32)
                    hi = lax.bitcast_convert_type(w & hi_mask, jnp.float32)
                    acc2[p][2 * h] = acc2[p][2 * h] + lo * m
                    acc2[p][2 * h + 1] = acc2[p][2 * h + 1] + hi * m
            accs = [acc2[0][d] + acc2[1][d] for d in range(D // LANES)]
            cnt = cnt2[0] + cnt2[1]
            denom = jnp.maximum(cnt, jnp.float32(1.0))
            for d in range(D // LANES):
                out_v[s][b, pl.ds(d * LANES, LANES)] = accs[d] / denom
            return inner

        lax.fori_loop(0, C, row, 0)
        pltpu.async_copy(out_v[s], out_hbm.at[pl.ds(base, C), :], osem[s])

    fetch(0, 0)

    def pair(p, carry):
        for s in (0, 1):
            ch = 2 * p + s
            nxt = ch + 1

            @pl.when(nxt < NCHUNK)
            def _():
                fetch(nxt, 1 - s)

            @pl.when(ch >= 2)
            def _():
                base_prev = row0 + (ch - 2) * C
                pltpu.make_async_copy(
                    out_v[s], out_hbm.at[pl.ds(base_prev, C), :], osem[s]
                ).wait()

            compute(ch, s)
        return carry

    lax.fori_loop(0, NCHUNK // 2, pair, 0)
    for s in (0, 1):
        base_last = row0 + (NCHUNK - 2 + s) * C
        pltpu.make_async_copy(
            out_v[s], out_hbm.at[pl.ds(base_last, C), :], osem[s]
        ).wait()


_kern = pl.kernel(
    _body,
    out_type=jax.ShapeDtypeStruct((B, D), jnp.float32),
    mesh=plsc.VectorSubcoreMesh(core_axis_name="c", subcore_axis_name="s"),
    compiler_params=pltpu.CompilerParams(use_tc_tiling_on_sc=False),
    scratch_types=[
        pltpu.VMEM((C * L,), jnp.int32),
        pltpu.VMEM((C * L,), jnp.int32),
        pltpu.VMEM((C, MPAD), jnp.float32),
        pltpu.VMEM((C, MPAD), jnp.float32),
        pltpu.VMEM((C * L, D // 2), jnp.int32),
        pltpu.VMEM((C * L, D // 2), jnp.int32),
        pltpu.VMEM((C, D), jnp.float32),
        pltpu.VMEM((C, D), jnp.float32),
        pltpu.SemaphoreType.DMA,
        pltpu.SemaphoreType.DMA,
        pltpu.SemaphoreType.DMA,
        pltpu.SemaphoreType.DMA,
    ],
)


V = 1000000
TW = 32768
QSH = (TW // 4).bit_length() - 1
NBLK = (V + TW - 1) // TW
VP = NBLK * TW
PW = D // 2


def _transpose_body(tt_ref, out_ref):
    x = tt_ref[...]
    q = [x[16 * i : 16 * (i + 1), :] for i in range(4)]
    qb = [
        lax.bitcast_convert_type(
            lax.bitcast_convert_type(qi.astype(jnp.bfloat16), jnp.uint16)
            .astype(jnp.uint32),
            jnp.int32,
        )
        for qi in q
    ]
    wa = qb[0] | (qb[1] << 16)
    wb = qb[2] | (qb[3] << 16)
    qt = TW // 4
    pieces = []
    for i in range(4):
        pieces.append(wa[:, i * qt : (i + 1) * qt])
        pieces.append(wb[:, i * qt : (i + 1) * qt])
    w = jnp.concatenate(pieces, axis=0)
    out_ref[...] = w.T


def _linearize_table(tt):
    return pl.pallas_call(
        _transpose_body,
        grid=(NBLK,),
        in_specs=[pl.BlockSpec((D, TW), lambda i: (0, i))],
        out_specs=pl.BlockSpec((TW // 4, 4 * PW), lambda i: (i, 0)),
        out_shape=jax.ShapeDtypeStruct((VP // 4, 4 * PW), jnp.int32),
    )(tt)


@jax.jit
def kernel(x, mask, table):
    mask_p = jnp.pad(mask, ((0, 0), (0, MPAD - L)))
    t_lin = _linearize_table(table.T).reshape(VP, PW)
    return _kern(x.reshape(-1), mask_p, t_lin)


# speedup vs baseline: 1.0087x; 1.0087x over previous
import functools

import jax
import jax.numpy as jnp
from jax import lax
from jax.experimental import pallas as pl
from jax.experimental.pallas import tpu as pltpu
from jax.experimental.pallas import tpu_sc as plsc

B = 16384
L = 50
D = 64
LANES = 16

_info = plsc.get_sparse_core_info()
NC = _info.num_cores
NS = _info.num_subcores
NW = NC * NS
ROWS_PER_W = B // NW
C = 16
NCHUNK = ROWS_PER_W // C


MPAD = 64


def _body(x_hbm, mask_hbm, table_hbm, out_hbm,
          idx_v0, idx_v1, mask_v0, mask_v1, rows_v0, rows_v1, out_v0, out_v1,
          gsem0, gsem1, osem0, osem1):
    wid = lax.axis_index("s") * NC + lax.axis_index("c")
    row0 = wid * ROWS_PER_W
    idx_v = (idx_v0, idx_v1)
    mask_v = (mask_v0, mask_v1)
    rows_v = (rows_v0, rows_v1)
    out_v = (out_v0, out_v1)
    gsem = (gsem0, gsem1)
    osem = (osem0, osem1)

    def fetch(ch, s):
        base = row0 + ch * C
        off = pl.multiple_of(base * L, 8)
        pltpu.sync_copy(x_hbm.at[pl.ds(off, C * L)], idx_v[s])
        pltpu.sync_copy(mask_hbm.at[pl.ds(base, C), :], mask_v[s])
        for k in range(C * L // LANES):
            v = idx_v[s][pl.ds(k * LANES, LANES)]
            o = v & (TW - 1)
            idx_v[s][pl.ds(k * LANES, LANES)] = (
                (v - o) + ((o & (TW // 4 - 1)) << 2) + (o >> QSH)
            )
        pltpu.async_copy(table_hbm.at[idx_v[s]], rows_v[s], gsem[s])

    def compute(ch, s):
        base = row0 + ch * C
        pltpu.make_async_copy(table_hbm.at[idx_v[s]], rows_v[s], gsem[s]).wait()

        hi_mask = jnp.full((LANES,), -65536, jnp.int32)

        def row_pair(i, inner):
            bs = (2 * i, 2 * i + 1)
            mvs = [
                [mask_v[s][b, pl.ds(k * LANES, LANES)] for k in range(MPAD // LANES)]
                for b in bs
            ]
            accs = [
                [jnp.zeros((LANES,), jnp.float32) for _ in range(D // LANES)]
                for _ in bs
            ]
            cnts = [jnp.float32(0.0), jnp.float32(0.0)]
            for l in range(L):
                for j, b in enumerate(bs):
                    m = mvs[j][l // LANES][l % LANES]
                    cnts[j] = cnts[j] + m
                    for h in range(PW // LANES):
                        w = rows_v[s][b * L + l, pl.ds(h * LANES, LANES)]
                        lo = lax.bitcast_convert_type(w << 16, jnp.float32)
                        hi = lax.bitcast_convert_type(w & hi_mask, jnp.float32)
                        accs[j][2 * h] = accs[j][2 * h] + lo * m
                        accs[j][2 * h + 1] = accs[j][2 * h + 1] + hi * m
            for j, b in enumerate(bs):
                denom = jnp.maximum(cnts[j], jnp.float32(1.0))
                for d in range(D // LANES):
                    out_v[s][b, pl.ds(d * LANES, LANES)] = accs[j][d] / denom
            return inner

        lax.fori_loop(0, C // 2, row_pair, 0)
        pltpu.async_copy(out_v[s], out_hbm.at[pl.ds(base, C), :], osem[s])

    fetch(0, 0)

    def pair(p, carry):
        for s in (0, 1):
            ch = 2 * p + s
            nxt = ch + 1

            @pl.when(nxt < NCHUNK)
            def _():
                fetch(nxt, 1 - s)

            @pl.when(ch >= 2)
            def _():
                base_prev = row0 + (ch - 2) * C
                pltpu.make_async_copy(
                    out_v[s], out_hbm.at[pl.ds(base_prev, C), :], osem[s]
                ).wait()

            compute(ch, s)
        return carry

    lax.fori_loop(0, NCHUNK // 2, pair, 0)
    for s in (0, 1):
        base_last = row0 + (NCHUNK - 2 + s) * C
        pltpu.make_async_copy(
            out_v[s], out_hbm.at[pl.ds(base_last, C), :], osem[s]
        ).wait()


_kern = pl.kernel(
    _body,
    out_type=jax.ShapeDtypeStruct((B, D), jnp.float32),
    mesh=plsc.VectorSubcoreMesh(core_axis_name="c", subcore_axis_name="s"),
    compiler_params=pltpu.CompilerParams(use_tc_tiling_on_sc=False),
    scratch_types=[
        pltpu.VMEM((C * L,), jnp.int32),
        pltpu.VMEM((C * L,), jnp.int32),
        pltpu.VMEM((C, MPAD), jnp.float32),
        pltpu.VMEM((C, MPAD), jnp.float32),
        pltpu.VMEM((C * L, D // 2), jnp.int32),
        pltpu.VMEM((C * L, D // 2), jnp.int32),
        pltpu.VMEM((C, D), jnp.float32),
        pltpu.VMEM((C, D), jnp.float32),
        pltpu.SemaphoreType.DMA,
        pltpu.SemaphoreType.DMA,
        pltpu.SemaphoreType.DMA,
        pltpu.SemaphoreType.DMA,
    ],
)


V = 1000000
TW = 32768
QSH = (TW // 4).bit_length() - 1
NBLK = (V + TW - 1) // TW
VP = NBLK * TW
PW = D // 2


def _transpose_body(tt_ref, out_ref):
    x = tt_ref[...]
    q = [x[16 * i : 16 * (i + 1), :] for i in range(4)]
    qb = [
        lax.bitcast_convert_type(
            lax.bitcast_convert_type(qi.astype(jnp.bfloat16), jnp.uint16)
            .astype(jnp.uint32),
            jnp.int32,
        )
        for qi in q
    ]
    wa = qb[0] | (qb[1] << 16)
    wb = qb[2] | (qb[3] << 16)
    qt = TW // 4
    pieces = []
    for i in range(4):
        pieces.append(wa[:, i * qt : (i + 1) * qt])
        pieces.append(wb[:, i * qt : (i + 1) * qt])
    w = jnp.concatenate(pieces, axis=0)
    out_ref[...] = w.T


def _linearize_table(tt):
    return pl.pallas_call(
        _transpose_body,
        grid=(NBLK,),
        in_specs=[pl.BlockSpec((D, TW), lambda i: (0, i))],
        out_specs=pl.BlockSpec((TW // 4, 4 * PW), lambda i: (i, 0)),
        out_shape=jax.ShapeDtypeStruct((VP // 4, 4 * PW), jnp.int32),
    )(tt)


@jax.jit
def kernel(x, mask, table):
    mask_p = jnp.pad(mask, ((0, 0), (0, MPAD - L)))
    t_lin = _linearize_table(table.T).reshape(VP, PW)
    return _kern(x.reshape(-1), mask_p, t_lin)


# speedup vs baseline: 1.0505x; 1.0414x over previous
import functools

import jax
import jax.numpy as jnp
from jax import lax
from jax.experimental import pallas as pl
from jax.experimental.pallas import tpu as pltpu
from jax.experimental.pallas import tpu_sc as plsc

B = 16384
L = 50
D = 64
LANES = 16

_info = plsc.get_sparse_core_info()
NC = _info.num_cores
NS = _info.num_subcores
NW = NC * NS
ROWS_PER_W = B // NW
C = 32
NCHUNK = ROWS_PER_W // C


MPAD = 64


def _body(x_hbm, mask_hbm, table_hbm, out_hbm,
          idx_v0, idx_v1, mask_v0, mask_v1, rows_v0, rows_v1, out_v0, out_v1,
          gsem0, gsem1, osem0, osem1):
    wid = lax.axis_index("s") * NC + lax.axis_index("c")
    row0 = wid * ROWS_PER_W
    idx_v = (idx_v0, idx_v1)
    mask_v = (mask_v0, mask_v1)
    rows_v = (rows_v0, rows_v1)
    out_v = (out_v0, out_v1)
    gsem = (gsem0, gsem1)
    osem = (osem0, osem1)

    def fetch(ch, s):
        base = row0 + ch * C
        off = pl.multiple_of(base * L, 8)
        pltpu.sync_copy(x_hbm.at[pl.ds(off, C * L)], idx_v[s])
        pltpu.sync_copy(mask_hbm.at[pl.ds(base, C), :], mask_v[s])
        for k in range(C * L // LANES):
            v = idx_v[s][pl.ds(k * LANES, LANES)]
            o = v & (TW - 1)
            idx_v[s][pl.ds(k * LANES, LANES)] = (
                (v - o) + ((o & (TW // 4 - 1)) << 2) + (o >> QSH)
            )
        pltpu.async_copy(table_hbm.at[idx_v[s]], rows_v[s], gsem[s])

    def compute(ch, s):
        base = row0 + ch * C
        pltpu.make_async_copy(table_hbm.at[idx_v[s]], rows_v[s], gsem[s]).wait()

        hi_mask = jnp.full((LANES,), -65536, jnp.int32)

        def row_pair(i, inner):
            bs = (2 * i, 2 * i + 1)
            mvs = [
                [mask_v[s][b, pl.ds(k * LANES, LANES)] for k in range(MPAD // LANES)]
                for b in bs
            ]
            accs = [
                [jnp.zeros((LANES,), jnp.float32) for _ in range(D // LANES)]
                for _ in bs
            ]
            cnts = [jnp.float32(0.0), jnp.float32(0.0)]
            for l in range(L):
                for j, b in enumerate(bs):
                    m = mvs[j][l // LANES][l % LANES]
                    cnts[j] = cnts[j] + m
                    for h in range(PW // LANES):
                        w = rows_v[s][b * L + l, pl.ds(h * LANES, LANES)]
                        lo = lax.bitcast_convert_type(w << 16, jnp.float32)
                        hi = lax.bitcast_convert_type(w & hi_mask, jnp.float32)
                        accs[j][2 * h] = accs[j][2 * h] + lo * m
                        accs[j][2 * h + 1] = accs[j][2 * h + 1] + hi * m
            for j, b in enumerate(bs):
                denom = jnp.maximum(cnts[j], jnp.float32(1.0))
                for d in range(D // LANES):
                    out_v[s][b, pl.ds(d * LANES, LANES)] = accs[j][d] / denom
            return inner

        lax.fori_loop(0, C // 2, row_pair, 0)
        pltpu.async_copy(out_v[s], out_hbm.at[pl.ds(base, C), :], osem[s])

    fetch(0, 0)

    def pair(p, carry):
        for s in (0, 1):
            ch = 2 * p + s
            nxt = ch + 1

            @pl.when(nxt < NCHUNK)
            def _():
                fetch(nxt, 1 - s)

            @pl.when(ch >= 2)
            def _():
                base_prev = row0 + (ch - 2) * C
                pltpu.make_async_copy(
                    out_v[s], out_hbm.at[pl.ds(base_prev, C), :], osem[s]
                ).wait()

            compute(ch, s)
        return carry

    lax.fori_loop(0, NCHUNK // 2, pair, 0)
    for s in (0, 1):
        base_last = row0 + (NCHUNK - 2 + s) * C
        pltpu.make_async_copy(
            out_v[s], out_hbm.at[pl.ds(base_last, C), :], osem[s]
        ).wait()


_kern = pl.kernel(
    _body,
    out_type=jax.ShapeDtypeStruct((B, D), jnp.float32),
    mesh=plsc.VectorSubcoreMesh(core_axis_name="c", subcore_axis_name="s"),
    compiler_params=pltpu.CompilerParams(use_tc_tiling_on_sc=False),
    scratch_types=[
        pltpu.VMEM((C * L,), jnp.int32),
        pltpu.VMEM((C * L,), jnp.int32),
        pltpu.VMEM((C, MPAD), jnp.float32),
        pltpu.VMEM((C, MPAD), jnp.float32),
        pltpu.VMEM((C * L, D // 2), jnp.int32),
        pltpu.VMEM((C * L, D // 2), jnp.int32),
        pltpu.VMEM((C, D), jnp.float32),
        pltpu.VMEM((C, D), jnp.float32),
        pltpu.SemaphoreType.DMA,
        pltpu.SemaphoreType.DMA,
        pltpu.SemaphoreType.DMA,
        pltpu.SemaphoreType.DMA,
    ],
)


V = 1000000
TW = 32768
QSH = (TW // 4).bit_length() - 1
NBLK = (V + TW - 1) // TW
VP = NBLK * TW
PW = D // 2


def _transpose_body(tt_ref, out_ref):
    x = tt_ref[...]
    q = [x[16 * i : 16 * (i + 1), :] for i in range(4)]
    qb = [
        lax.bitcast_convert_type(
            lax.bitcast_convert_type(qi.astype(jnp.bfloat16), jnp.uint16)
            .astype(jnp.uint32),
            jnp.int32,
        )
        for qi in q
    ]
    wa = qb[0] | (qb[1] << 16)
    wb = qb[2] | (qb[3] << 16)
    qt = TW // 4
    pieces = []
    for i in range(4):
        pieces.append(wa[:, i * qt : (i + 1) * qt])
        pieces.append(wb[:, i * qt : (i + 1) * qt])
    w = jnp.concatenate(pieces, axis=0)
    out_ref[...] = w.T


def _linearize_table(tt):
    return pl.pallas_call(
        _transpose_body,
        grid=(NBLK,),
        in_specs=[pl.BlockSpec((D, TW), lambda i: (0, i))],
        out_specs=pl.BlockSpec((TW // 4, 4 * PW), lambda i: (i, 0)),
        out_shape=jax.ShapeDtypeStruct((VP // 4, 4 * PW), jnp.int32),
    )(tt)


@jax.jit
def kernel(x, mask, table):
    mask_p = jnp.pad(mask, ((0, 0), (0, MPAD - L)))
    t_lin = _linearize_table(table.T).reshape(VP, PW)
    return _kern(x.reshape(-1), mask_p, t_lin)


# speedup vs baseline: 1.0515x; 1.0009x over previous
import jax
import jax.numpy as jnp
from jax import lax
from jax.experimental import pallas as pl
from jax.experimental.pallas import tpu as pltpu
from jax.experimental.pallas import tpu_sc as plsc

B = 16384
L = 50
D = 64
LANES = 16

_info = plsc.get_sparse_core_info()
NC = _info.num_cores
NS = _info.num_subcores
NW = NC * NS
ROWS_PER_W = B // NW
C = 32
NCHUNK = ROWS_PER_W // C


MPAD = 64


def _body(x_hbm, mask_hbm, table_hbm, out_hbm,
          idx_v0, idx_v1, mask_v0, mask_v1, rows_v0, rows_v1, out_v0, out_v1,
          gsem0, gsem1, osem0, osem1):
    wid = lax.axis_index("s") * NC + lax.axis_index("c")
    row0 = wid * ROWS_PER_W
    idx_v = (idx_v0, idx_v1)
    mask_v = (mask_v0, mask_v1)
    rows_v = (rows_v0, rows_v1)
    out_v = (out_v0, out_v1)
    gsem = (gsem0, gsem1)
    osem = (osem0, osem1)

    def fetch(ch, s):
        base = row0 + ch * C
        off = pl.multiple_of(base * L, 8)
        pltpu.sync_copy(x_hbm.at[pl.ds(off, C * L)], idx_v[s])
        pltpu.sync_copy(mask_hbm.at[pl.ds(base, C), :], mask_v[s])
        for k in range(C * L // LANES):
            v = idx_v[s][pl.ds(k * LANES, LANES)]
            o = v & (TW - 1)
            idx_v[s][pl.ds(k * LANES, LANES)] = (
                (v - o) + ((o & (TW // 4 - 1)) << 2) + (o >> QSH)
            )
        pltpu.async_copy(table_hbm.at[idx_v[s]], rows_v[s], gsem[s])

    def compute(ch, s):
        base = row0 + ch * C
        pltpu.make_async_copy(table_hbm.at[idx_v[s]], rows_v[s], gsem[s]).wait()

        hi_mask = jnp.full((LANES,), -65536, jnp.int32)

        def row_pair(i, inner):
            bs = (2 * i, 2 * i + 1)
            mvs = [
                [mask_v[s][b, pl.ds(k * LANES, LANES)] for k in range(MPAD // LANES)]
                for b in bs
            ]
            accs = [
                [jnp.zeros((LANES,), jnp.float32) for _ in range(D // LANES)]
                for _ in bs
            ]
            cnts = [jnp.float32(0.0), jnp.float32(0.0)]
            for l in range(L):
                for j, b in enumerate(bs):
                    m = mvs[j][l // LANES][l % LANES]
                    cnts[j] = cnts[j] + m
                    for h in range(PW // LANES):
                        w = rows_v[s][b * L + l, pl.ds(h * LANES, LANES)]
                        lo = lax.bitcast_convert_type(w << 16, jnp.float32)
                        hi = lax.bitcast_convert_type(w & hi_mask, jnp.float32)
                        accs[j][2 * h] = accs[j][2 * h] + lo * m
                        accs[j][2 * h + 1] = accs[j][2 * h + 1] + hi * m
            for j, b in enumerate(bs):
                denom = jnp.maximum(cnts[j], jnp.float32(1.0))
                for d in range(D // LANES):
                    out_v[s][b, pl.ds(d * LANES, LANES)] = accs[j][d] / denom
            return inner

        lax.fori_loop(0, C // 2, row_pair, 0)
        pltpu.async_copy(out_v[s], out_hbm.at[pl.ds(base, C), :], osem[s])

    fetch(0, 0)

    def pair(p, carry):
        for s in (0, 1):
            ch = 2 * p + s
            nxt = ch + 1

            @pl.when(nxt < NCHUNK)
            def _():
                fetch(nxt, 1 - s)

            @pl.when(ch >= 2)
            def _():
                base_prev = row0 + (ch - 2) * C
                pltpu.make_async_copy(
                    out_v[s], out_hbm.at[pl.ds(base_prev, C), :], osem[s]
                ).wait()

            compute(ch, s)
        return carry

    lax.fori_loop(0, NCHUNK // 2, pair, 0)
    for s in (0, 1):
        base_last = row0 + (NCHUNK - 2 + s) * C
        pltpu.make_async_copy(
            out_v[s], out_hbm.at[pl.ds(base_last, C), :], osem[s]
        ).wait()


_kern = pl.kernel(
    _body,
    out_type=jax.ShapeDtypeStruct((B, D), jnp.float32),
    mesh=plsc.VectorSubcoreMesh(core_axis_name="c", subcore_axis_name="s"),
    compiler_params=pltpu.CompilerParams(use_tc_tiling_on_sc=False),
    scratch_types=[
        pltpu.VMEM((C * L,), jnp.int32),
        pltpu.VMEM((C * L,), jnp.int32),
        pltpu.VMEM((C, MPAD), jnp.float32),
        pltpu.VMEM((C, MPAD), jnp.float32),
        pltpu.VMEM((C * L, D // 2), jnp.int32),
        pltpu.VMEM((C * L, D // 2), jnp.int32),
        pltpu.VMEM((C, D), jnp.float32),
        pltpu.VMEM((C, D), jnp.float32),
        pltpu.SemaphoreType.DMA,
        pltpu.SemaphoreType.DMA,
        pltpu.SemaphoreType.DMA,
        pltpu.SemaphoreType.DMA,
    ],
)


V = 1000000
TW = 32768
QSH = (TW // 4).bit_length() - 1
NBLK = (V + TW - 1) // TW
VP = NBLK * TW
PW = D // 2


def _transpose_body(tt_ref, out_ref):
    x = tt_ref[...]
    q = [x[16 * i : 16 * (i + 1), :] for i in range(4)]
    qb = [
        lax.bitcast_convert_type(
            lax.bitcast_convert_type(qi.astype(jnp.bfloat16), jnp.uint16)
            .astype(jnp.uint32),
            jnp.int32,
        )
        for qi in q
    ]
    wa = qb[0] | (qb[1] << 16)
    wb = qb[2] | (qb[3] << 16)
    qt = TW // 4
    pieces = []
    for i in range(4):
        pieces.append(wa[:, i * qt : (i + 1) * qt])
        pieces.append(wb[:, i * qt : (i + 1) * qt])
    w = jnp.concatenate(pieces, axis=0)
    out_ref[...] = w.T


def _linearize_table(tt):
    return pl.pallas_call(
        _transpose_body,
        grid=(NBLK,),
        in_specs=[pl.BlockSpec((D, TW), lambda i: (0, i))],
        out_specs=pl.BlockSpec((TW // 4, 4 * PW), lambda i: (i, 0)),
        out_shape=jax.ShapeDtypeStruct((VP // 4, 4 * PW), jnp.int32),
    )(tt)


@jax.jit
def kernel(x, mask, table):
    mask_p = jnp.pad(mask, ((0, 0), (0, MPAD - L)))
    t_lin = _linearize_table(table.T).reshape(VP, PW)
    return _kern(x.reshape(-1), mask_p, t_lin)
